# baseline (speedup 1.0000x reference)
import functools

import jax
import jax.numpy as jnp
from jax import lax
from jax.experimental import pallas as pl
from jax.experimental.pallas import tpu as pltpu
from jax.experimental.pallas import tpu_sc as plsc

VOCAB = 65
VP = 80
EMB = 32
B, T = 32, 2048
NC, NS = 2, 16
NW = NC * NS
CH = T // NW
CHV = CH * VOCAB
DEPTH = 4
OB = 2


def _pre_body(tok_ref, pos_ref, w_ref, b_ref, tokl_ref, posl_ref):
    w = w_ref[...]
    zt = jnp.zeros((VOCAB, VP - VOCAB), jnp.float32)
    zp = jnp.zeros((T, VP - VOCAB), jnp.float32)
    tokl = (
        jnp.dot(tok_ref[...], w, preferred_element_type=jnp.float32)
        + b_ref[0][None, :]
    )
    tokl_ref[...] = jnp.concatenate([tokl, zt], axis=1)
    posl = jnp.dot(pos_ref[...], w, preferred_element_type=jnp.float32)
    posl_ref[...] = jnp.concatenate([posl, zp], axis=1)


def _precompute(tok_table, pos_table, W, b):
    return pl.pallas_call(
        _pre_body,
        out_shape=[
            jax.ShapeDtypeStruct((VOCAB, VP), jnp.float32),
            jax.ShapeDtypeStruct((T, VP), jnp.float32),
        ],
    )(tok_table, pos_table, W, b.reshape(1, VOCAB))


@functools.partial(
    pl.kernel,
    out_type=jax.ShapeDtypeStruct((B * T * VOCAB,), jnp.float32),
    mesh=plsc.VectorSubcoreMesh(core_axis_name="c", subcore_axis_name="s"),
    scratch_types=[
        pltpu.VMEM((B * CH,), jnp.int32),
        pltpu.VMEM((CH, VP), jnp.float32),
        [pltpu.VMEM((CH, VP), jnp.float32) for _ in range(DEPTH)],
        [pltpu.VMEM((CHV + 16,), jnp.float32) for _ in range(OB)],
        pltpu.SemaphoreType.DMA,
        pltpu.SemaphoreType.DMA,
    ],
    compiler_params=pltpu.CompilerParams(
        use_tc_tiling_on_sc=False, needs_layout_passes=False
    ),
)
def _sc_gather_add(idx_hbm, tokl_hbm, posl_hbm, out_hbm,
                   idx_v, pos_v, bufs, obufs, gsem, osem):
    wid = lax.axis_index("s") * NC + lax.axis_index("c")
    t0 = wid * CH
    pltpu.sync_copy(posl_hbm.at[pl.ds(t0, CH), :], pos_v)
    for bb in range(B):
        pltpu.sync_copy(
            idx_hbm.at[pl.ds(bb * T + t0, CH)], idx_v.at[pl.ds(bb * CH, CH)]
        )

    def gather(j):
        return pltpu.async_copy(
            tokl_hbm.at[idx_v.at[pl.ds(j * CH, CH)]], bufs[j % DEPTH], gsem
        )

    handles = {j: gather(j) for j in range(DEPTH)}
    out_handles = {}
    ibase = lax.iota(jnp.int32, 16)
    for j in range(B):
        handles.pop(j).wait()
        buf = bufs[j % DEPTH]
        obuf = obufs[j % OB]
        if j - OB in out_handles:
            out_handles.pop(j - OB).wait()

        def row_body(r, _, buf=buf, obuf=obuf):
            for u in range(2):
                rr = 2 * r + u
                base = rr * VOCAB
                for k in range(VP // 16):
                    v = buf[rr, pl.ds(16 * k, 16)] + pos_v[rr, pl.ds(16 * k, 16)]
                    plsc.store_scatter(obuf, [ibase + (base + 16 * k)], v)
            return 0

        lax.fori_loop(0, CH // 2, row_body, 0)
        out_handles[j] = pltpu.async_copy(
            obuf.at[pl.ds(0, CHV)],
            out_hbm.at[pl.ds(j * (T * VOCAB) + t0 * VOCAB, CHV)],
            osem,
        )
        if j + DEPTH < B:
            handles[j + DEPTH] = gather(j + DEPTH)
    for j in sorted(out_handles):
        out_handles.pop(j).wait()


def kernel(idx, tok_table, pos_table, W, b):
    tokl, posl = _precompute(tok_table, pos_table, W, b)
    out = _sc_gather_add(idx.reshape(B * T), tokl, posl)
    return out.reshape(B, T, VOCAB)


# speedup vs baseline: 1.0190x; 1.0190x over previous
import functools

import jax
import jax.numpy as jnp
from jax import lax
from jax.experimental import pallas as pl
from jax.experimental.pallas import tpu as pltpu
from jax.experimental.pallas import tpu_sc as plsc

VOCAB = 65
VP = 80
EMB = 32
B, T = 32, 2048
NC, NS = 2, 16
NW = NC * NS
CH = T // NW
CHV = CH * VOCAB
DEPTH = 4
OB = 2
NK = VP // 16
UR = 4


def _pre_body(tok_ref, pos_ref, w_ref, b_ref, tokl_ref, posl_ref):
    w = w_ref[...]
    zt = jnp.zeros((VOCAB, VP - VOCAB), jnp.float32)
    zp = jnp.zeros((T, VP - VOCAB), jnp.float32)
    tokl = (
        jnp.dot(tok_ref[...], w, preferred_element_type=jnp.float32)
        + b_ref[0][None, :]
    )
    tokl_ref[...] = jnp.concatenate([tokl, zt], axis=1)
    posl = jnp.dot(pos_ref[...], w, preferred_element_type=jnp.float32)
    posl_ref[...] = jnp.concatenate([posl, zp], axis=1)


def _precompute(tok_table, pos_table, W, b):
    return pl.pallas_call(
        _pre_body,
        out_shape=[
            jax.ShapeDtypeStruct((VOCAB, VP), jnp.float32),
            jax.ShapeDtypeStruct((T, VP), jnp.float32),
        ],
    )(tok_table, pos_table, W, b.reshape(1, VOCAB))


@functools.partial(
    pl.kernel,
    out_type=jax.ShapeDtypeStruct((B * T * VOCAB,), jnp.float32),
    mesh=plsc.VectorSubcoreMesh(core_axis_name="c", subcore_axis_name="s"),
    scratch_types=[
        pltpu.VMEM((B * CH,), jnp.int32),
        pltpu.VMEM((CH, VP), jnp.float32),
        [pltpu.VMEM((CH, VP), jnp.float32) for _ in range(DEPTH)],
        [pltpu.VMEM((CHV,), jnp.float32) for _ in range(OB)],
        pltpu.SemaphoreType.DMA,
        pltpu.SemaphoreType.DMA,
    ],
    compiler_params=pltpu.CompilerParams(
        use_tc_tiling_on_sc=False, needs_layout_passes=False
    ),
)
def _sc_gather_add(idx_hbm, tokl_hbm, posl_hbm, out_hbm,
                   idx_v, pos_v, bufs, obufs, gsem, osem):
    wid = lax.axis_index("s") * NC + lax.axis_index("c")
    t0 = wid * CH
    pltpu.sync_copy(posl_hbm.at[pl.ds(t0, CH), :], pos_v)
    for bb in range(B):
        pltpu.sync_copy(
            idx_hbm.at[pl.ds(bb * T + t0, CH)], idx_v.at[pl.ds(bb * CH, CH)]
        )

    def gather(j):
        return pltpu.async_copy(
            tokl_hbm.at[idx_v.at[pl.ds(j * CH, CH)]], bufs[j % DEPTH], gsem
        )

    handles = {j: gather(j) for j in range(DEPTH)}
    out_handles = {}
    iota = lax.iota(jnp.int32, 16)
    cols = [iota + 16 * k for k in range(4)] + [iota * 0 + 64]
    m64 = iota < 1
    for j in range(B):
        handles.pop(j).wait()
        buf = bufs[j % DEPTH]
        obuf = obufs[j % OB]
        if j - OB in out_handles:
            out_handles.pop(j - OB).wait()

        def row_body(r, _, buf=buf, obuf=obuf):
            rows = [iota * 0 + ((UR * r + u) * VOCAB) for u in range(UR)]
            vals = [
                [
                    buf[UR * r + u, pl.ds(16 * k, 16)]
                    + pos_v[UR * r + u, pl.ds(16 * k, 16)]
                    for k in range(NK)
                ]
                for u in range(UR)
            ]
            for u in range(UR):
                for k in range(4):
                    plsc.store_scatter(obuf, [rows[u] + cols[k]], vals[u][k])
                plsc.store_scatter(
                    obuf, [rows[u] + cols[4]], vals[u][4], mask=m64
                )
            return 0

        lax.fori_loop(0, CH // UR, row_body, 0)
        out_handles[j] = pltpu.async_copy(
            obuf,
            out_hbm.at[pl.ds(j * (T * VOCAB) + t0 * VOCAB, CHV)],
            osem,
        )
        if j + DEPTH < B:
            handles[j + DEPTH] = gather(j + DEPTH)
    for j in sorted(out_handles):
        out_handles.pop(j).wait()


def kernel(idx, tok_table, pos_table, W, b):
    tokl, posl = _precompute(tok_table, pos_table, W, b)
    out = _sc_gather_add(idx.reshape(B * T), tokl, posl)
    return out.reshape(B, T, VOCAB)


# speedup vs baseline: 1.3626x; 1.3372x over previous
import functools

import jax
import jax.numpy as jnp
from jax import lax
from jax.experimental import pallas as pl
from jax.experimental.pallas import tpu as pltpu
from jax.experimental.pallas import tpu_sc as plsc

VOCAB = 65
VP = 80
EMB = 32
B, T = 32, 2048
NC, NS = 2, 16
NW = NC * NS
CH = T // NW
CHV = CH * VOCAB
DEPTH = 4
OB = 2
NK = VP // 16
UR = 4


def _pre_body(tok_ref, pos_ref, w_ref, b_ref, tokl_ref, posl_ref):
    w = w_ref[...]
    zt = jnp.zeros((VOCAB, VP - VOCAB), jnp.float32)
    zp = jnp.zeros((T, VP - VOCAB), jnp.float32)
    tokl = (
        jnp.dot(tok_ref[...], w, preferred_element_type=jnp.float32)
        + b_ref[0][None, :]
    )
    tokl_ref[...] = jnp.concatenate([tokl, zt], axis=1)
    posl = jnp.dot(pos_ref[...], w, preferred_element_type=jnp.float32)
    posl_ref[...] = jnp.concatenate([posl, zp], axis=1)


def _precompute(tok_table, pos_table, W, b):
    return pl.pallas_call(
        _pre_body,
        out_shape=[
            jax.ShapeDtypeStruct((VOCAB, VP), jnp.float32),
            jax.ShapeDtypeStruct((T, VP), jnp.float32),
        ],
    )(tok_table, pos_table, W, b.reshape(1, VOCAB))


@functools.partial(
    pl.kernel,
    out_type=jax.ShapeDtypeStruct((B * T * VOCAB,), jnp.float32),
    mesh=plsc.VectorSubcoreMesh(core_axis_name="c", subcore_axis_name="s"),
    scratch_types=[
        pltpu.VMEM((B * CH,), jnp.int32),
        pltpu.VMEM((CH, VP), jnp.float32),
        [pltpu.VMEM((CH, VP), jnp.float32) for _ in range(DEPTH)],
        [pltpu.VMEM((CHV,), jnp.float32) for _ in range(OB)],
        pltpu.VMEM((VOCAB, VP), jnp.float32),
        pltpu.VMEM_SHARED((VOCAB, VP), jnp.float32),
        pltpu.SemaphoreType.DMA,
        pltpu.SemaphoreType.DMA,
    ],
    compiler_params=pltpu.CompilerParams(
        use_tc_tiling_on_sc=False, needs_layout_passes=False
    ),
)
def _sc_gather_add(idx_hbm, tokl_hbm, posl_hbm, out_hbm,
                   idx_v, pos_v, bufs, obufs, tokl_v, tokl_spm, gsem, osem):
    sid = lax.axis_index("s")
    wid = sid * NC + lax.axis_index("c")
    t0 = wid * CH

    @pl.when(sid == 0)
    def _():
        pltpu.sync_copy(tokl_hbm, tokl_v)
        pltpu.sync_copy(tokl_v, tokl_spm)

    pltpu.sync_copy(posl_hbm.at[pl.ds(t0, CH), :], pos_v)
    for bb in range(B):
        pltpu.sync_copy(
            idx_hbm.at[pl.ds(bb * T + t0, CH)], idx_v.at[pl.ds(bb * CH, CH)]
        )
    plsc.subcore_barrier()

    def gather(j):
        return pltpu.async_copy(
            tokl_spm.at[idx_v.at[pl.ds(j * CH, CH)]], bufs[j % DEPTH], gsem
        )

    handles = {j: gather(j) for j in range(DEPTH)}
    out_handles = {}
    iota = lax.iota(jnp.int32, 16)
    cols = [iota + 16 * k for k in range(4)] + [iota * 0 + 64]
    m64 = iota < 1
    for j in range(B):
        handles.pop(j).wait()
        buf = bufs[j % DEPTH]
        obuf = obufs[j % OB]
        if j - OB in out_handles:
            out_handles.pop(j - OB).wait()

        def row_body(r, _, buf=buf, obuf=obuf):
            rows = [iota * 0 + ((UR * r + u) * VOCAB) for u in range(UR)]
            vals = [
                [
                    buf[UR * r + u, pl.ds(16 * k, 16)]
                    + pos_v[UR * r + u, pl.ds(16 * k, 16)]
                    for k in range(NK)
                ]
                for u in range(UR)
            ]
            for u in range(UR):
                for k in range(4):
                    plsc.store_scatter(obuf, [rows[u] + cols[k]], vals[u][k])
                plsc.store_scatter(
                    obuf, [rows[u] + cols[4]], vals[u][4], mask=m64
                )
            return 0

        lax.fori_loop(0, CH // UR, row_body, 0)
        out_handles[j] = pltpu.async_copy(
            obuf,
            out_hbm.at[pl.ds(j * (T * VOCAB) + t0 * VOCAB, CHV)],
            osem,
        )
        if j + DEPTH < B:
            handles[j + DEPTH] = gather(j + DEPTH)
    for j in sorted(out_handles):
        out_handles.pop(j).wait()


def kernel(idx, tok_table, pos_table, W, b):
    tokl, posl = _precompute(tok_table, pos_table, W, b)
    out = _sc_gather_add(idx.reshape(B * T), tokl, posl)
    return out.reshape(B, T, VOCAB)


# speedup vs baseline: 2.0191x; 1.4817x over previous
import functools

import jax
import jax.numpy as jnp
from jax import lax
from jax.experimental import pallas as pl
from jax.experimental.pallas import tpu as pltpu
from jax.experimental.pallas import tpu_sc as plsc

VOCAB = 65
VP = 128
EMB = 32
B, T = 32, 2048
NC, NS = 2, 16
NW = NC * NS
CH = T // NW
DEPTH = 4
OB = 2
UR = 4
TAIL = VOCAB - 16


def _pre_body(tok_ref, pos_ref, w_ref, b_ref, tokl_ref, posl_ref):
    w = w_ref[...]
    zt = jnp.zeros((VOCAB, VP - VOCAB), jnp.float32)
    zp = jnp.zeros((T, VP - VOCAB), jnp.float32)
    tokl = (
        jnp.dot(tok_ref[...], w, preferred_element_type=jnp.float32)
        + b_ref[0][None, :]
    )
    tokl_ref[...] = jnp.concatenate([tokl, zt], axis=1)
    posl = jnp.dot(pos_ref[...], w, preferred_element_type=jnp.float32)
    posl_ref[...] = jnp.concatenate([posl, zp], axis=1)


def _precompute(tok_table, pos_table, W, b):
    return pl.pallas_call(
        _pre_body,
        out_shape=[
            jax.ShapeDtypeStruct((VOCAB, VP), jnp.float32),
            jax.ShapeDtypeStruct((T, VP), jnp.float32),
        ],
    )(tok_table, pos_table, W, b.reshape(1, VOCAB))


@functools.partial(
    pl.kernel,
    out_type=jax.ShapeDtypeStruct((B, T, VOCAB), jnp.float32),
    mesh=plsc.VectorSubcoreMesh(core_axis_name="c", subcore_axis_name="s"),
    scratch_types=[
        pltpu.VMEM((B * CH,), jnp.int32),
        pltpu.VMEM((CH, VP), jnp.float32),
        [pltpu.VMEM((CH, VP), jnp.float32) for _ in range(DEPTH)],
        [pltpu.VMEM((CH, VOCAB), jnp.float32) for _ in range(OB)],
        pltpu.VMEM((VOCAB, VP), jnp.float32),
        pltpu.VMEM_SHARED((VOCAB, VP), jnp.float32),
        pltpu.SemaphoreType.DMA,
        pltpu.SemaphoreType.DMA,
    ],
    compiler_params=pltpu.CompilerParams(needs_layout_passes=False),
)
def _sc_gather_add(idx_hbm, tokl_hbm, posl_hbm, out_hbm,
                   idx_v, pos_v, bufs, obufs, tokl_v, tokl_spm, gsem, osem):
    sid = lax.axis_index("s")
    cid = lax.axis_index("c")
    wid = sid * NC + cid
    t0 = wid * CH

    @pl.when(sid == 0)
    def _():
        pltpu.sync_copy(tokl_hbm, tokl_v)
        pltpu.sync_copy(tokl_v, tokl_spm)

    pltpu.sync_copy(posl_hbm.at[pl.ds(t0, CH), :], pos_v)
    irow = wid // NC
    icol = (wid % NC) * CH
    for bb in range(B):
        pltpu.sync_copy(
            idx_hbm.at[16 * bb + irow, pl.ds(icol, CH)],
            idx_v.at[pl.ds(bb * CH, CH)],
        )
    plsc.subcore_barrier()

    def gather(j):
        return pltpu.async_copy(
            tokl_spm.at[idx_v.at[pl.ds(j * CH, CH)]], bufs[j % DEPTH], gsem
        )

    handles = {j: gather(j) for j in range(DEPTH)}
    out_handles = {}
    for j in range(B):
        handles.pop(j).wait()
        buf = bufs[j % DEPTH]
        obuf = obufs[j % OB]
        if j - OB in out_handles:
            out_handles.pop(j - OB).wait()

        def row_body(r, _, buf=buf, obuf=obuf):
            for u in range(UR):
                rr = UR * r + u
                for k in range(4):
                    obuf[rr, pl.ds(16 * k, 16)] = (
                        buf[rr, pl.ds(16 * k, 16)]
                        + pos_v[rr, pl.ds(16 * k, 16)]
                    )
                obuf[rr, pl.ds(TAIL, 16)] = (
                    buf[rr, pl.ds(TAIL, 16)] + pos_v[rr, pl.ds(TAIL, 16)]
                )
            return 0

        lax.fori_loop(0, CH // UR, row_body, 0)
        out_handles[j] = pltpu.async_copy(
            obuf, out_hbm.at[j, pl.ds(t0, CH), :], osem
        )
        if j + DEPTH < B:
            handles[j + DEPTH] = gather(j + DEPTH)
    for j in sorted(out_handles):
        out_handles.pop(j).wait()


def kernel(idx, tok_table, pos_table, W, b):
    tokl, posl = _precompute(tok_table, pos_table, W, b)
    return _sc_gather_add(idx.reshape(B * T // VP, VP), tokl, posl)


# speedup vs baseline: 2.1782x; 1.0788x over previous
import functools

import jax
import jax.numpy as jnp
from jax import lax
from jax.experimental import pallas as pl
from jax.experimental.pallas import tpu as pltpu
from jax.experimental.pallas import tpu_sc as plsc

VOCAB = 65
VP = 128
EMB = 32
B, T = 32, 2048
NC, NS = 2, 16
NW = NC * NS
CH = T // NW
DEPTH = 4
OB = 2
UR = 4
TAIL = VOCAB - 16


def _pre_body(tok_ref, pos_ref, w_ref, b_ref, tokl_ref, posl_ref):
    w = w_ref[...]
    zt = jnp.zeros((VOCAB, VP - VOCAB), jnp.float32)
    zp = jnp.zeros((T, VP - VOCAB), jnp.float32)
    tokl = (
        jnp.dot(tok_ref[...], w, preferred_element_type=jnp.float32)
        + b_ref[0][None, :]
    )
    tokl_ref[...] = jnp.concatenate([tokl, zt], axis=1)
    posl = jnp.dot(pos_ref[...], w, preferred_element_type=jnp.float32)
    posl_ref[...] = jnp.concatenate([posl, zp], axis=1)


def _precompute(tok_table, pos_table, W, b):
    return pl.pallas_call(
        _pre_body,
        out_shape=[
            jax.ShapeDtypeStruct((VOCAB, VP), jnp.float32),
            jax.ShapeDtypeStruct((T, VP), jnp.float32),
        ],
    )(tok_table, pos_table, W, b.reshape(1, VOCAB))


@functools.partial(
    pl.kernel,
    out_type=jax.ShapeDtypeStruct((B * T, VOCAB), jnp.float32),
    mesh=plsc.VectorSubcoreMesh(core_axis_name="c", subcore_axis_name="s"),
    scratch_types=[
        pltpu.VMEM((B * CH,), jnp.int32),
        pltpu.VMEM((CH, VP), jnp.float32),
        [pltpu.VMEM((CH, VP), jnp.float32) for _ in range(DEPTH)],
        [pltpu.VMEM((CH, VOCAB), jnp.float32) for _ in range(OB)],
        pltpu.VMEM((VOCAB, VP), jnp.float32),
        pltpu.VMEM_SHARED((VOCAB, VP), jnp.float32),
        pltpu.SemaphoreType.DMA,
        pltpu.SemaphoreType.DMA,
    ],
    compiler_params=pltpu.CompilerParams(needs_layout_passes=False),
)
def _sc_gather_add(idx_hbm, tokl_hbm, posl_hbm, out_hbm,
                   idx_v, pos_v, bufs, obufs, tokl_v, tokl_spm, gsem, osem):
    sid = lax.axis_index("s")
    cid = lax.axis_index("c")
    wid = sid * NC + cid
    t0 = wid * CH

    @pl.when(sid == 0)
    def _():
        pltpu.sync_copy(tokl_hbm, tokl_v)
        pltpu.sync_copy(tokl_v, tokl_spm)

    pltpu.sync_copy(posl_hbm.at[pl.ds(t0, CH), :], pos_v)
    irow = wid // NC
    icol = (wid % NC) * CH
    for bb in range(B):
        pltpu.sync_copy(
            idx_hbm.at[16 * bb + irow, pl.ds(icol, CH)],
            idx_v.at[pl.ds(bb * CH, CH)],
        )
    plsc.subcore_barrier()

    def gather(j):
        return pltpu.async_copy(
            tokl_spm.at[idx_v.at[pl.ds(j * CH, CH)]], bufs[j % DEPTH], gsem
        )

    handles = {j: gather(j) for j in range(DEPTH)}
    out_handles = {}
    for j in range(B):
        handles.pop(j).wait()
        buf = bufs[j % DEPTH]
        obuf = obufs[j % OB]
        if j - OB in out_handles:
            out_handles.pop(j - OB).wait()

        def row_body(r, _, buf=buf, obuf=obuf):
            for u in range(UR):
                rr = UR * r + u
                for k in range(4):
                    obuf[rr, pl.ds(16 * k, 16)] = (
                        buf[rr, pl.ds(16 * k, 16)]
                        + pos_v[rr, pl.ds(16 * k, 16)]
                    )
                obuf[rr, pl.ds(TAIL, 16)] = (
                    buf[rr, pl.ds(TAIL, 16)] + pos_v[rr, pl.ds(TAIL, 16)]
                )
            return 0

        lax.fori_loop(0, CH // UR, row_body, 0)
        out_handles[j] = pltpu.async_copy(
            obuf, out_hbm.at[pl.ds(j * T + t0, CH), :], osem
        )
        if j + DEPTH < B:
            handles[j + DEPTH] = gather(j + DEPTH)
    for j in sorted(out_handles):
        out_handles.pop(j).wait()


def kernel(idx, tok_table, pos_table, W, b):
    tokl, posl = _precompute(tok_table, pos_table, W, b)
    out = _sc_gather_add(idx.reshape(B * T // VP, VP), tokl, posl)
    return out.reshape(B, T, VOCAB)
